# baseline (speedup 1.0000x reference)
import jax
import jax.numpy as jnp
from jax import lax
from jax.experimental import pallas as pl
from jax.experimental.pallas import tpu as pltpu
from jax.experimental.pallas import tpu_sc as plsc

BATCH = 4
MAX_LEN = 2048
EMBED = 1024
NC, NS, L = 2, 16, 16
NW = NC * NS
T_PER_W = MAX_LEN // NW
CT = 16
NQ = T_PER_W // CT
NPAIR = 2
NU = NQ * NPAIR
NBUF = 3
VREGS_PER_ROW = EMBED // L


def _body(seq_hbm, tok_hbm, pos_hbm, out_hbm, idx_v,
          rows0, rows1, rows2, pos_v,
          gsem0, gsem1, gsem2, psem, ssem0, ssem1, ssem2):
    wid = lax.axis_index("s") * NC + lax.axis_index("c")
    tw0 = wid * T_PER_W
    for b in range(BATCH):
        pltpu.sync_copy(seq_hbm.at[b, pl.ds(tw0, T_PER_W)], idx_v.at[b])

    rows = [rows0, rows1, rows2]
    gsem = [gsem0, gsem1, gsem2]
    ssem = [ssem0, ssem1, ssem2]

    def start_pos(q):
        return [pltpu.async_copy(pos_hbm.at[pl.ds(tw0 + q * CT, CT)],
                                 pos_v, psem)]

    def start_unit(u):
        q, pr = divmod(u, NPAIR)
        rb = u % NBUF
        return [
            pltpu.async_copy(
                tok_hbm.at[idx_v.at[2 * pr + i, pl.ds(q * CT, CT)]],
                rows[rb].at[i], gsem[rb])
            for i in range(2)
        ]

    pend_pos = {0: start_pos(0)}
    pend_g = {0: start_unit(0), 1: start_unit(1)}
    pend_s = {}
    for u in range(NU):
        q, pr = divmod(u, NPAIR)
        rb = u % NBUF
        nxt = u + 2
        if nxt < NU:
            nb = nxt % NBUF
            for d in pend_s.pop(nb, ()):
                d.wait()
            pend_g[nxt] = start_unit(nxt)
        for d in pend_g.pop(u):
            d.wait()
        if pr == 0:
            for d in pend_pos.pop(q):
                d.wait()

        def add_j(j, carry, _rb=rb):
            sl = pl.ds(j * L, L)
            for r in range(CT):
                p = pos_v[r, sl]
                for i in range(2):
                    rows[_rb][i, r, sl] = rows[_rb][i, r, sl] + p
            return carry

        lax.fori_loop(0, VREGS_PER_ROW, add_j, 0)

        t0 = tw0 + q * CT
        pend_s[rb] = [
            pltpu.async_copy(rows[rb].at[i], out_hbm.at[2 * pr + i, pl.ds(t0, CT)],
                             ssem[rb])
            for i in range(2)
        ]
        if pr == NPAIR - 1 and q + 1 < NQ:
            pend_pos[q + 1] = start_pos(q + 1)
    for descs in pend_s.values():
        for d in descs:
            d.wait()


def kernel(sequence, token_weight, position_weight):
    mesh = plsc.VectorSubcoreMesh(core_axis_name="c", subcore_axis_name="s")
    f = pl.kernel(
        _body,
        out_type=jax.ShapeDtypeStruct((BATCH, MAX_LEN, EMBED), jnp.float32),
        mesh=mesh,
        scratch_types=[
            pltpu.VMEM((BATCH, T_PER_W), jnp.int32),
            pltpu.VMEM((NPAIR, CT, EMBED), jnp.float32),
            pltpu.VMEM((NPAIR, CT, EMBED), jnp.float32),
            pltpu.VMEM((NPAIR, CT, EMBED), jnp.float32),
            pltpu.VMEM((CT, EMBED), jnp.float32),
            pltpu.SemaphoreType.DMA,
            pltpu.SemaphoreType.DMA,
            pltpu.SemaphoreType.DMA,
            pltpu.SemaphoreType.DMA,
            pltpu.SemaphoreType.DMA,
            pltpu.SemaphoreType.DMA,
            pltpu.SemaphoreType.DMA,
        ],
    )
    return f(sequence, token_weight, position_weight)
